# baseline (speedup 1.0000x reference)
import functools

import jax
import jax.numpy as jnp
from jax import lax
from jax.experimental import pallas as pl
from jax.experimental.pallas import tpu as pltpu
from jax.experimental.pallas import tpu_sc as plsc

NC = 2
NS = 16
NW = NC * NS
RING = 16


@functools.partial(jax.jit, static_argnums=(2, 3))
def _gather_rows(table, idx, n, d):
    b_per_w = n // NW

    mesh = plsc.VectorSubcoreMesh(core_axis_name="c", subcore_axis_name="s")

    @functools.partial(
        pl.kernel,
        mesh=mesh,
        out_type=jax.ShapeDtypeStruct((n, d), jnp.float32),
        scratch_types=[
            pltpu.VMEM((b_per_w,), jnp.int32),
            [pltpu.SemaphoreType.DMA for _ in range(RING)],
            pltpu.SemaphoreType.DMA,
        ],
    )
    def k(table_hbm, idx_hbm, out_hbm, idx_v, sems, isem):
        wid = lax.axis_index("s") * NC + lax.axis_index("c")
        base = wid * b_per_w
        pltpu.sync_copy(idx_hbm.at[pl.ds(base, b_per_w)], idx_v)

        def group_copy(p):
            vec = idx_v[pl.ds(pl.multiple_of(p * RING, RING), RING)]
            for j in range(RING):
                row = vec[j]
                pltpu.async_copy(
                    table_hbm.at[pl.ds(row, 1)],
                    out_hbm.at[pl.ds(base + p * RING + j, 1)],
                    sems[j],
                )

        def wait_group():
            for j in range(RING):
                pltpu.make_async_copy(
                    table_hbm.at[pl.ds(0, 1)],
                    out_hbm.at[pl.ds(base, 1)],
                    sems[j],
                ).wait()

        group_copy(0)

        def ring_body(p, carry):
            wait_group()
            group_copy(p)
            return carry

        lax.fori_loop(1, b_per_w // RING, ring_body, 0)

        wait_group()

    return k(table, idx)


def kernel(input, embed_weight):
    b, t = input.shape
    v, d = embed_weight.shape
    idx = input.reshape(b * t).astype(jnp.int32)
    out = _gather_rows(embed_weight, idx, b * t, d)
    return out.reshape(b, t, d)


# speedup vs baseline: 36.4430x; 36.4430x over previous
import functools

import jax
import jax.numpy as jnp
from jax import lax
from jax.experimental import pallas as pl
from jax.experimental.pallas import tpu as pltpu
from jax.experimental.pallas import tpu_sc as plsc

NC = 2
NS = 16
NW = NC * NS

CHUNK = 8


@functools.partial(jax.jit, static_argnums=(2, 3))
def _gather_rows(table, idx, n, d):
    b_per_w = n // NW
    n_chunks = b_per_w // CHUNK

    mesh = plsc.VectorSubcoreMesh(core_axis_name="c", subcore_axis_name="s")

    @functools.partial(
        pl.kernel,
        mesh=mesh,
        out_type=jax.ShapeDtypeStruct((n, d), jnp.float32),
        scratch_types=[
            pltpu.VMEM((b_per_w,), jnp.int32),
            pltpu.VMEM((CHUNK, d), jnp.float32),
            pltpu.VMEM((CHUNK, d), jnp.float32),
            pltpu.SemaphoreType.DMA,
            pltpu.SemaphoreType.DMA,
            pltpu.SemaphoreType.DMA,
            pltpu.SemaphoreType.DMA,
        ],
    )
    def k(table_hbm, idx_hbm, out_hbm, idx_v, buf0, buf1,
          gsem0, gsem1, ssem0, ssem1):
        wid = lax.axis_index("s") * NC + lax.axis_index("c")
        base = wid * b_per_w
        pltpu.sync_copy(idx_hbm.at[pl.ds(base, b_per_w)], idx_v)

        bufs = (buf0, buf1)
        gsems = (gsem0, gsem1)
        ssems = (ssem0, ssem1)

        def start_gather(g, b):
            off = pl.multiple_of(g * CHUNK, CHUNK)
            return pltpu.async_copy(
                table_hbm.at[idx_v.at[pl.ds(off, CHUNK)]], bufs[b], gsems[b]
            )

        def start_scatter(g, b):
            off = pl.multiple_of(g * CHUNK, CHUNK)
            return pltpu.async_copy(
                bufs[b], out_hbm.at[pl.ds(base + off, CHUNK)], ssems[b]
            )

        def wait_scatter(b):
            pltpu.make_async_copy(
                bufs[b], out_hbm.at[pl.ds(base, CHUNK)], ssems[b]
            ).wait()

        def wait_gather(b):
            pltpu.make_async_copy(
                table_hbm.at[idx_v.at[pl.ds(0, CHUNK)]], bufs[b], gsems[b]
            ).wait()

        start_gather(0, 0)
        start_gather(1, 1)
        wait_gather(0)
        start_scatter(0, 0)

        def pair_body(p, carry):
            for b in range(2):
                g = p * 2 + b
                wait_scatter(b)
                start_gather(g, b)
                wait_gather(1 - b)
                start_scatter(g - 1, 1 - b)
            return carry

        lax.fori_loop(1, n_chunks // 2, pair_body, 0)

        wait_gather(1)
        start_scatter(n_chunks - 1, 1)
        wait_scatter(0)
        wait_scatter(1)

    return k(table, idx)


def kernel(input, embed_weight):
    b, t = input.shape
    v, d = embed_weight.shape
    idx = input.reshape(b * t).astype(jnp.int32)
    out = _gather_rows(embed_weight, idx, b * t, d)
    return out.reshape(b, t, d)
